# baseline (speedup 1.0000x reference)
import functools

import jax
import jax.numpy as jnp
from jax import lax
from jax.experimental import pallas as pl
from jax.experimental.pallas import tpu as pltpu
from jax.experimental.pallas import tpu_sc as plsc

N = 10000
D = 128
K = 32
NC = 2
NS = 16
NW = NC * NS
CHUNK = 80
NCHUNKS = N // CHUNK
CPW = -(-NCHUNKS // NW)

BLK = 2000


def _sc_gather_sum(x, adjb):
  mesh = plsc.VectorSubcoreMesh(
      core_axis_name="c", subcore_axis_name="s", num_cores=NC, num_subcores=NS)

  @functools.partial(
      pl.kernel,
      out_type=jax.ShapeDtypeStruct((N, D), jnp.float32),
      mesh=mesh,
      scratch_types=[
          pltpu.VMEM((K, CHUNK), jnp.int32),
          pltpu.VMEM((CHUNK, D), jnp.float32),
          pltpu.SemaphoreType.DMA,
      ],
  )
  def sc_kernel(x_hbm, adjb_hbm, out_hbm, idx_v, acc_v, sem):
    wid = lax.axis_index("s") * NC + lax.axis_index("c")
    for ci in range(CPW):
      c = wid * CPW + ci
      @pl.when(c < NCHUNKS)
      def _():
        pltpu.sync_copy(adjb_hbm.at[c], idx_v)
        pltpu.async_copy(x_hbm.at[idx_v.at[0]], acc_v, sem).wait()

        def fire(kk, carry):
          pltpu.async_copy(x_hbm.at[idx_v.at[kk]], acc_v, sem, add=True)
          return carry
        lax.fori_loop(1, K, fire, 0)

        def drain(kk, carry):
          pltpu.make_async_copy(x_hbm.at[pl.ds(0, CHUNK)], acc_v, sem).wait()
          return carry
        lax.fori_loop(1, K, drain, 0)

        pltpu.sync_copy(acc_v, out_hbm.at[pl.ds(c * CHUNK, CHUNK)])

  return sc_kernel(x, adjb)


def _tc_combine(x, s, e2, Wc, WnK, M, b2):
  def body(x_ref, s_ref, e_ref, wc_ref, wn_ref, m_ref, b_ref, o_ref):
    acc = jnp.dot(x_ref[...], wc_ref[...],
                  preferred_element_type=jnp.float32)
    acc += jnp.dot(s_ref[...].astype(jnp.bfloat16), wn_ref[...],
                   preferred_element_type=jnp.float32)
    acc += jnp.dot(e_ref[...], m_ref[...],
                   preferred_element_type=jnp.float32)
    o_ref[...] = jnp.maximum(acc + b_ref[...], 0.0)

  return pl.pallas_call(
      body,
      grid=(N // BLK,),
      in_specs=[
          pl.BlockSpec((BLK, D), lambda i: (i, 0)),
          pl.BlockSpec((BLK, D), lambda i: (i, 0)),
          pl.BlockSpec((BLK, 2 * K), lambda i: (i, 0)),
          pl.BlockSpec((D, D), lambda i: (0, 0)),
          pl.BlockSpec((D, D), lambda i: (0, 0)),
          pl.BlockSpec((2 * K, D), lambda i: (0, 0)),
          pl.BlockSpec((1, D), lambda i: (0, 0)),
      ],
      out_specs=pl.BlockSpec((BLK, D), lambda i: (i, 0)),
      out_shape=jax.ShapeDtypeStruct((N, D), jnp.float32),
      compiler_params=pltpu.CompilerParams(
          dimension_semantics=("arbitrary",)),
  )(x, s, e2, Wc, WnK, M, b2)


def kernel(x, adj, edge, Wc, Wn, We, q, b, training):
  del q, training
  adjb = (adj.astype(jnp.int32)
          .reshape(NCHUNKS, CHUNK, K).transpose(0, 2, 1))
  s = _sc_gather_sum(x, adjb)
  bf = jnp.bfloat16
  xb = x.astype(bf)
  e2b = edge.reshape(N, 2 * K).astype(bf)
  inv_k = jnp.float32(1.0 / K)
  WnKb = (Wn * inv_k).astype(bf)
  Mb = (jnp.tile(We, (K, 1)) * inv_k).astype(bf)
  b2 = b.reshape(1, D)
  return _tc_combine(xb, s, e2b, Wc.astype(bf), WnKb, Mb, b2)


# speedup vs baseline: 1.0196x; 1.0196x over previous
import functools

import jax
import jax.numpy as jnp
from jax import lax
from jax.experimental import pallas as pl
from jax.experimental.pallas import tpu as pltpu
from jax.experimental.pallas import tpu_sc as plsc

N = 10000
D = 128
K = 32
NC = 2
NS = 16
NW = NC * NS
CHUNK = 80
NCHUNKS = N // CHUNK
CPW = -(-NCHUNKS // NW)

BLK = 2000


def _sc_gather_sum(x, adjb):
  mesh = plsc.VectorSubcoreMesh(
      core_axis_name="c", subcore_axis_name="s", num_cores=NC, num_subcores=NS)

  @functools.partial(
      pl.kernel,
      out_type=jax.ShapeDtypeStruct((N, D), jnp.float32),
      mesh=mesh,
      scratch_types=[
          pltpu.VMEM((K, CHUNK), jnp.int32),
          pltpu.VMEM((CHUNK, D), jnp.float32),
          pltpu.SemaphoreType.DMA,
      ],
  )
  def sc_kernel(x_hbm, adjb_hbm, out_hbm, idx_v, acc_v, sem):
    wid = lax.axis_index("s") * NC + lax.axis_index("c")
    for ci in range(CPW):
      c = wid * CPW + ci
      @pl.when(c < NCHUNKS)
      def _():
        pltpu.sync_copy(adjb_hbm.at[c], idx_v)
        pltpu.async_copy(x_hbm.at[idx_v.at[0]], acc_v, sem).wait()

        def fire(kk, carry):
          pltpu.async_copy(x_hbm.at[idx_v.at[kk]], acc_v, sem, add=True)
          return carry
        lax.fori_loop(1, K, fire, 0)

        def drain(kk, carry):
          pltpu.make_async_copy(x_hbm.at[pl.ds(0, CHUNK)], acc_v, sem).wait()
          return carry
        lax.fori_loop(1, K, drain, 0)

        pltpu.sync_copy(acc_v, out_hbm.at[pl.ds(c * CHUNK, CHUNK)])

  return sc_kernel(x, adjb)


def _tc_combine(x, s, e2, Wc, WnK, M, b2):
  def body(x_ref, s_ref, e_ref, wc_ref, wn_ref, m_ref, b_ref, o_ref):
    bf = jnp.bfloat16
    acc = jnp.dot(x_ref[...].astype(bf), wc_ref[...].astype(bf),
                  preferred_element_type=jnp.float32)
    acc += jnp.dot(s_ref[...].astype(bf), wn_ref[...].astype(bf),
                   preferred_element_type=jnp.float32)
    acc += jnp.dot(e_ref[...].astype(bf), m_ref[...].astype(bf),
                   preferred_element_type=jnp.float32)
    o_ref[...] = jnp.maximum(acc + b_ref[...], 0.0)

  return pl.pallas_call(
      body,
      grid=(N // BLK,),
      in_specs=[
          pl.BlockSpec((BLK, D), lambda i: (i, 0)),
          pl.BlockSpec((BLK, D), lambda i: (i, 0)),
          pl.BlockSpec((BLK, 2 * K), lambda i: (i, 0)),
          pl.BlockSpec((D, D), lambda i: (0, 0)),
          pl.BlockSpec((D, D), lambda i: (0, 0)),
          pl.BlockSpec((2 * K, D), lambda i: (0, 0)),
          pl.BlockSpec((1, D), lambda i: (0, 0)),
      ],
      out_specs=pl.BlockSpec((BLK, D), lambda i: (i, 0)),
      out_shape=jax.ShapeDtypeStruct((N, D), jnp.float32),
      compiler_params=pltpu.CompilerParams(
          dimension_semantics=("arbitrary",)),
  )(x, s, e2, Wc, WnK, M, b2)


def kernel(x, adj, edge, Wc, Wn, We, q, b, training):
  del q, training
  adjb = (adj.astype(jnp.int32)
          .reshape(NCHUNKS, CHUNK, K).transpose(0, 2, 1))
  s = _sc_gather_sum(x, adjb)
  e2 = edge.reshape(N, 2 * K)
  inv_k = jnp.float32(1.0 / K)
  WnK = Wn * inv_k
  M = jnp.tile(We, (K, 1)) * inv_k
  b2 = b.reshape(1, D)
  return _tc_combine(x, s, e2, Wc, WnK, M, b2)


# speedup vs baseline: 1.0436x; 1.0235x over previous
import functools

import jax
import jax.numpy as jnp
from jax import lax
from jax.experimental import pallas as pl
from jax.experimental.pallas import tpu as pltpu
from jax.experimental.pallas import tpu_sc as plsc

N = 10000
D = 128
K = 32
NC = 2
NS = 16
NW = NC * NS
CHUNK = 80
NCHUNKS = N // CHUNK
CPW = -(-NCHUNKS // NW)

BLK = 2000


def _sc_gather_sum(x, adjb):
  mesh = plsc.VectorSubcoreMesh(
      core_axis_name="c", subcore_axis_name="s", num_cores=NC, num_subcores=NS)

  @functools.partial(
      pl.kernel,
      out_type=jax.ShapeDtypeStruct((N, D), jnp.float32),
      mesh=mesh,
      scratch_types=[
          pltpu.VMEM((K, CHUNK), jnp.int32),
          pltpu.VMEM((CHUNK, D), jnp.float32),
          pltpu.SemaphoreType.DMA,
          pltpu.SemaphoreType.DMA,
      ],
  )
  def sc_kernel(x_hbm, adjb_hbm, out_hbm, idx_v, acc_v, sem, sidx):
    wid = lax.axis_index("s") * NC + lax.axis_index("c")
    zz = jnp.zeros((16,), jnp.float32)
    for ci in range(CPW):
      c = wid * CPW + ci
      @pl.when(c < NCHUNKS)
      def _():
        pltpu.async_copy(adjb_hbm.at[c], idx_v, sidx)
        def zrow(r, carry):
          for w in range(D // 16):
            acc_v[r, pl.ds(16 * w, 16)] = zz
          return carry
        lax.fori_loop(0, CHUNK, zrow, 0)
        pltpu.make_async_copy(adjb_hbm.at[c], idx_v, sidx).wait()

        def fire(kk, carry):
          pltpu.async_copy(x_hbm.at[idx_v.at[kk]], acc_v, sem, add=True)
          return carry
        lax.fori_loop(0, K, fire, 0)

        def drain(kk, carry):
          pltpu.make_async_copy(x_hbm.at[pl.ds(0, CHUNK)], acc_v, sem).wait()
          return carry
        lax.fori_loop(0, K, drain, 0)

        pltpu.sync_copy(acc_v, out_hbm.at[pl.ds(c * CHUNK, CHUNK)])

  return sc_kernel(x, adjb)


def _tc_combine(x, s, e2, Wc, WnK, M, b2):
  def body(x_ref, s_ref, e_ref, wc_ref, wn_ref, m_ref, b_ref, o_ref):
    bf = jnp.bfloat16
    acc = jnp.dot(x_ref[...].astype(bf), wc_ref[...].astype(bf),
                  preferred_element_type=jnp.float32)
    acc += jnp.dot(s_ref[...].astype(bf), wn_ref[...].astype(bf),
                   preferred_element_type=jnp.float32)
    acc += jnp.dot(e_ref[...].astype(bf), m_ref[...].astype(bf),
                   preferred_element_type=jnp.float32)
    o_ref[...] = jnp.maximum(acc + b_ref[...], 0.0)

  return pl.pallas_call(
      body,
      grid=(N // BLK,),
      in_specs=[
          pl.BlockSpec((BLK, D), lambda i: (i, 0)),
          pl.BlockSpec((BLK, D), lambda i: (i, 0)),
          pl.BlockSpec((BLK, 2 * K), lambda i: (i, 0)),
          pl.BlockSpec((D, D), lambda i: (0, 0)),
          pl.BlockSpec((D, D), lambda i: (0, 0)),
          pl.BlockSpec((2 * K, D), lambda i: (0, 0)),
          pl.BlockSpec((1, D), lambda i: (0, 0)),
      ],
      out_specs=pl.BlockSpec((BLK, D), lambda i: (i, 0)),
      out_shape=jax.ShapeDtypeStruct((N, D), jnp.float32),
      compiler_params=pltpu.CompilerParams(
          dimension_semantics=("arbitrary",)),
  )(x, s, e2, Wc, WnK, M, b2)


def kernel(x, adj, edge, Wc, Wn, We, q, b, training):
  del q, training
  adjb = (adj.astype(jnp.int32)
          .reshape(NCHUNKS, CHUNK, K).transpose(0, 2, 1))
  s = _sc_gather_sum(x, adjb)
  e2 = edge.reshape(N, 2 * K)
  inv_k = jnp.float32(1.0 / K)
  WnK = Wn * inv_k
  M = jnp.tile(We, (K, 1)) * inv_k
  b2 = b.reshape(1, D)
  return _tc_combine(x, s, e2, Wc, WnK, M, b2)


# speedup vs baseline: 1.0557x; 1.0116x over previous
import functools

import jax
import jax.numpy as jnp
from jax import lax
from jax.experimental import pallas as pl
from jax.experimental.pallas import tpu as pltpu
from jax.experimental.pallas import tpu_sc as plsc

N = 10000
D = 128
K = 32
NC = 2
NS = 16
NW = NC * NS
CHUNK = 80
NCHUNKS = N // CHUNK
CPW = -(-NCHUNKS // NW)

BLK = 2000


def _sc_gather_sum(x, adjb):
  mesh = plsc.VectorSubcoreMesh(
      core_axis_name="c", subcore_axis_name="s", num_cores=NC, num_subcores=NS)

  @functools.partial(
      pl.kernel,
      out_type=jax.ShapeDtypeStruct((N, D), jnp.float32),
      mesh=mesh,
      scratch_types=[
          pltpu.VMEM((K, CHUNK), jnp.int32),
          pltpu.VMEM((CHUNK, D), jnp.float32),
          pltpu.VMEM((CHUNK, D), jnp.float32),
          pltpu.SemaphoreType.DMA,
          pltpu.SemaphoreType.DMA,
          pltpu.SemaphoreType.DMA,
          pltpu.SemaphoreType.DMA,
      ],
  )
  def sc_kernel(x_hbm, adjb_hbm, out_hbm, idx_v, acc0, acc1,
                sem, sidx, swb0, swb1):
    acc = (acc0, acc1)
    swb = (swb0, swb1)
    wid = lax.axis_index("s") * NC + lax.axis_index("c")
    zz = jnp.zeros((16,), jnp.float32)
    for ci in range(CPW):
      p = ci % 2
      c = wid * CPW + ci
      @pl.when(c < NCHUNKS)
      def _():
        pltpu.async_copy(adjb_hbm.at[c], idx_v, sidx)
        if ci >= 2:
          pltpu.make_async_copy(x_hbm.at[pl.ds(0, CHUNK)],
                                acc[p], swb[p]).wait()
        def zrow(r, carry):
          for w in range(D // 16):
            acc[p][r, pl.ds(16 * w, 16)] = zz
          return carry
        lax.fori_loop(0, CHUNK, zrow, 0)
        pltpu.make_async_copy(adjb_hbm.at[c], idx_v, sidx).wait()

        def fire(kk, carry):
          pltpu.async_copy(x_hbm.at[idx_v.at[kk]], acc[p], sem, add=True)
          return carry
        lax.fori_loop(0, K, fire, 0)

        def drain(kk, carry):
          pltpu.make_async_copy(x_hbm.at[pl.ds(0, CHUNK)], acc[p], sem).wait()
          return carry
        lax.fori_loop(0, K, drain, 0)

        pltpu.async_copy(acc[p], out_hbm.at[pl.ds(c * CHUNK, CHUNK)], swb[p])

    pltpu.make_async_copy(x_hbm.at[pl.ds(0, CHUNK)], acc[0], swb[0]).wait()
    @pl.when(wid * CPW + 1 < NCHUNKS)
    def _():
      pltpu.make_async_copy(x_hbm.at[pl.ds(0, CHUNK)], acc[1], swb[1]).wait()

  return sc_kernel(x, adjb)


def _tc_combine(x, s, e2, Wc, WnK, M, b2):
  def body(x_ref, s_ref, e_ref, wc_ref, wn_ref, m_ref, b_ref, o_ref):
    bf = jnp.bfloat16
    acc = jnp.dot(x_ref[...].astype(bf), wc_ref[...].astype(bf),
                  preferred_element_type=jnp.float32)
    acc += jnp.dot(s_ref[...].astype(bf), wn_ref[...].astype(bf),
                   preferred_element_type=jnp.float32)
    acc += jnp.dot(e_ref[...].astype(bf), m_ref[...].astype(bf),
                   preferred_element_type=jnp.float32)
    o_ref[...] = jnp.maximum(acc + b_ref[...], 0.0)

  return pl.pallas_call(
      body,
      grid=(N // BLK,),
      in_specs=[
          pl.BlockSpec((BLK, D), lambda i: (i, 0)),
          pl.BlockSpec((BLK, D), lambda i: (i, 0)),
          pl.BlockSpec((BLK, 2 * K), lambda i: (i, 0)),
          pl.BlockSpec((D, D), lambda i: (0, 0)),
          pl.BlockSpec((D, D), lambda i: (0, 0)),
          pl.BlockSpec((2 * K, D), lambda i: (0, 0)),
          pl.BlockSpec((1, D), lambda i: (0, 0)),
      ],
      out_specs=pl.BlockSpec((BLK, D), lambda i: (i, 0)),
      out_shape=jax.ShapeDtypeStruct((N, D), jnp.float32),
      compiler_params=pltpu.CompilerParams(
          dimension_semantics=("arbitrary",)),
  )(x, s, e2, Wc, WnK, M, b2)


def kernel(x, adj, edge, Wc, Wn, We, q, b, training):
  del q, training
  adjb = (adj.astype(jnp.int32)
          .reshape(NCHUNKS, CHUNK, K).transpose(0, 2, 1))
  s = _sc_gather_sum(x, adjb)
  e2 = edge.reshape(N, 2 * K)
  inv_k = jnp.float32(1.0 / K)
  WnK = Wn * inv_k
  M = jnp.tile(We, (K, 1)) * inv_k
  b2 = b.reshape(1, D)
  return _tc_combine(x, s, e2, Wc, WnK, M, b2)


# speedup vs baseline: 1.0808x; 1.0237x over previous
import functools

import jax
import jax.numpy as jnp
from jax import lax
from jax.experimental import pallas as pl
from jax.experimental.pallas import tpu as pltpu
from jax.experimental.pallas import tpu_sc as plsc

N = 10000
D = 128
K = 32
NC = 2
NS = 16
NW = NC * NS
CHUNK = 80
NCHUNKS = N // CHUNK
CPW = -(-NCHUNKS // NW)

BLK = 2000


def _sc_gather_sum(x, adjb):
  mesh = plsc.VectorSubcoreMesh(
      core_axis_name="c", subcore_axis_name="s", num_cores=NC, num_subcores=NS)

  @functools.partial(
      pl.kernel,
      out_type=jax.ShapeDtypeStruct((N, D), jnp.float32),
      mesh=mesh,
      scratch_types=[
          pltpu.VMEM((K, CHUNK), jnp.int32),
          pltpu.VMEM((K, CHUNK), jnp.int32),
          pltpu.VMEM((CHUNK, D), jnp.float32),
          pltpu.VMEM((CHUNK, D), jnp.float32),
          pltpu.SemaphoreType.DMA,
          pltpu.SemaphoreType.DMA,
          pltpu.SemaphoreType.DMA,
          pltpu.SemaphoreType.DMA,
          pltpu.SemaphoreType.DMA,
          pltpu.SemaphoreType.DMA,
      ],
  )
  def sc_kernel(x_hbm, adjb_hbm, out_hbm, idx0, idx1, acc0, acc1,
                sidx0, sidx1, sadd0, sadd1, swb0, swb1):
    idx = (idx0, idx1)
    acc = (acc0, acc1)
    sidx = (sidx0, sidx1)
    sadd = (sadd0, sadd1)
    swb = (swb0, swb1)
    wid = lax.axis_index("s") * NC + lax.axis_index("c")
    zz = jnp.zeros((16,), jnp.float32)

    def fire_idx(c, p):
      pltpu.async_copy(adjb_hbm.at[c], idx[p], sidx[p])

    def wait_idx(p):
      pltpu.make_async_copy(adjb_hbm.at[0], idx[p], sidx[p]).wait()

    def zero(p):
      def zrow(r, carry):
        for w in range(D // 16):
          acc[p][r, pl.ds(16 * w, 16)] = zz
        return carry
      lax.fori_loop(0, CHUNK, zrow, 0)

    def fire_adds(p):
      def fire(kk, carry):
        pltpu.async_copy(x_hbm.at[idx[p].at[kk]], acc[p], sadd[p], add=True)
        return carry
      lax.fori_loop(0, K, fire, 0)

    def drain_adds(p):
      def drain(kk, carry):
        pltpu.make_async_copy(x_hbm.at[pl.ds(0, CHUNK)], acc[p], sadd[p]).wait()
        return carry
      lax.fori_loop(0, K, drain, 0)

    def fire_wb(c, p):
      pltpu.async_copy(acc[p], out_hbm.at[pl.ds(c * CHUNK, CHUNK)], swb[p])

    def wait_wb(p):
      pltpu.make_async_copy(x_hbm.at[pl.ds(0, CHUNK)], acc[p], swb[p]).wait()

    c0 = wid * CPW
    fire_idx(c0, 0)
    for ci in range(CPW):
      p = ci % 2
      c = c0 + ci
      @pl.when(c < NCHUNKS)
      def _():
        if ci >= 2:
          wait_wb(p)
        zero(p)
        wait_idx(p)
        fire_adds(p)
      if ci >= 1:
        @pl.when(c - 1 < NCHUNKS)
        def _():
          drain_adds(1 - p)
          fire_wb(c - 1, 1 - p)
      if ci + 1 < CPW:
        @pl.when(c + 1 < NCHUNKS)
        def _():
          fire_idx(c + 1, 1 - p)

    p_last = (CPW - 1) % 2
    @pl.when(c0 + CPW - 1 < NCHUNKS)
    def _():
      drain_adds(p_last)
      fire_wb(c0 + CPW - 1, p_last)
    wait_wb(0)
    @pl.when(c0 + 1 < NCHUNKS)
    def _():
      wait_wb(1)

  return sc_kernel(x, adjb)


def _tc_combine(x, s, e2, Wc, WnK, M, b2):
  def body(x_ref, s_ref, e_ref, wc_ref, wn_ref, m_ref, b_ref, o_ref):
    bf = jnp.bfloat16
    acc = jnp.dot(x_ref[...].astype(bf), wc_ref[...].astype(bf),
                  preferred_element_type=jnp.float32)
    acc += jnp.dot(s_ref[...].astype(bf), wn_ref[...].astype(bf),
                   preferred_element_type=jnp.float32)
    acc += jnp.dot(e_ref[...].astype(bf), m_ref[...].astype(bf),
                   preferred_element_type=jnp.float32)
    o_ref[...] = jnp.maximum(acc + b_ref[...], 0.0)

  return pl.pallas_call(
      body,
      grid=(N // BLK,),
      in_specs=[
          pl.BlockSpec((BLK, D), lambda i: (i, 0)),
          pl.BlockSpec((BLK, D), lambda i: (i, 0)),
          pl.BlockSpec((BLK, 2 * K), lambda i: (i, 0)),
          pl.BlockSpec((D, D), lambda i: (0, 0)),
          pl.BlockSpec((D, D), lambda i: (0, 0)),
          pl.BlockSpec((2 * K, D), lambda i: (0, 0)),
          pl.BlockSpec((1, D), lambda i: (0, 0)),
      ],
      out_specs=pl.BlockSpec((BLK, D), lambda i: (i, 0)),
      out_shape=jax.ShapeDtypeStruct((N, D), jnp.float32),
      compiler_params=pltpu.CompilerParams(
          dimension_semantics=("arbitrary",)),
  )(x, s, e2, Wc, WnK, M, b2)


def kernel(x, adj, edge, Wc, Wn, We, q, b, training):
  del q, training
  adjb = (adj.astype(jnp.int32)
          .reshape(NCHUNKS, CHUNK, K).transpose(0, 2, 1))
  s = _sc_gather_sum(x, adjb)
  e2 = edge.reshape(N, 2 * K)
  inv_k = jnp.float32(1.0 / K)
  WnK = Wn * inv_k
  M = jnp.tile(We, (K, 1)) * inv_k
  b2 = b.reshape(1, D)
  return _tc_combine(x, s, e2, Wc, WnK, M, b2)
